# baseline (speedup 1.0000x reference)
import jax
import jax.numpy as jnp
from jax.experimental import pallas as pl

VN = 128
VD = 32
GRID = 2.0
VS = GRID / VN
XMIN = -GRID / 2.0
XMAX = GRID / 2.0
L_DIR = 4
B = 1024
M = 3 * (VN + 1)


def _posenc(d):
    outs = [d]
    for l in range(L_DIR):
        outs.append(jnp.sin((2.0 ** l) * d))
        outs.append(jnp.cos((2.0 ** l) * d))
    return jnp.concatenate(outs, axis=-1)


def _mlp(x, ws, bs):
    for i in range(len(ws) - 1):
        x = jax.nn.relu(x @ ws[i] + bs[i])
    return x @ ws[-1] + bs[-1]


def _trilerp(voxels, p):
    p = jnp.clip(p, 0.0, float(VN))
    p0 = jnp.floor(jnp.clip(p, 0.0, VN - 1e-5)).astype(jnp.int32)
    f = p - p0.astype(p.dtype)
    out = 0.0
    for dx in (0, 1):
        wx = f[..., 0] if dx else 1.0 - f[..., 0]
        for dy in (0, 1):
            wy = f[..., 1] if dy else 1.0 - f[..., 1]
            for dz in (0, 1):
                wz = f[..., 2] if dz else 1.0 - f[..., 2]
                v = voxels[p0[..., 0] + dx, p0[..., 1] + dy, p0[..., 2] + dz]
                out = out + (wx * wy * wz)[..., None] * v
    return out


def _mask_kernel(color_ref, sigma_ref, mask3_ref, mask2_ref, color_out, sigma_out):
    color_out[...] = color_ref[...] * mask3_ref[...]
    sigma_out[...] = sigma_ref[...] * mask2_ref[...]


def kernel(os, ds, voxels, w1_0, b1_0, w1_1, b1_1, w1_2, b1_2, w2_0, b2_0, w2_1, b2_1, w2_2, b2_2):
    eps = 1e-9
    d_safe = jnp.where(jnp.abs(ds) < eps, eps, ds)
    planes = XMIN + jnp.arange(VN + 1, dtype=jnp.float32) * VS
    t = (planes[None, None, :] - os[:, :, None]) / d_safe[:, :, None]
    t = t.reshape(os.shape[0], -1)
    t0 = (XMIN - os) / d_safe
    t1 = (XMAX - os) / d_safe
    tmin = jnp.maximum(jnp.max(jnp.minimum(t0, t1), axis=1), 0.0)
    tmax = jnp.min(jnp.maximum(t0, t1), axis=1)
    BIG = 1e10
    valid = (t >= tmin[:, None]) & (t <= tmax[:, None]) & (tmax > tmin)[:, None]
    t = jnp.where(valid, t, BIG)
    ts = jnp.sort(t, axis=1)
    mpts = ts < BIG * 0.5
    coord = (os[:, None, :] + ts[:, :, None] * ds[:, None, :] - XMIN) / VS
    mask = mpts[:, :-1] & mpts[:, 1:]
    cin = jnp.clip(coord[:, :-1], 0.0, float(VN))
    cout = jnp.clip(coord[:, 1:], 0.0, float(VN))
    cmid = 0.5 * (cin + cout)
    seg = jnp.sqrt(jnp.sum(((cout - cin) * VS) ** 2, axis=-1) + 1e-12)
    feat = (seg[..., None] / 6.0) * (_trilerp(voxels, cin) + 4.0 * _trilerp(voxels, cmid) + _trilerp(voxels, cout))
    feat = feat * mask[..., None]
    x = _mlp(feat, [w1_0, w1_1, w1_2], [b1_0, b1_1, b1_2])
    sigma_ = x[..., 0]
    h = x[..., 1:]
    venc = _posenc(ds)
    venc_b = jnp.broadcast_to(venc[:, None, :], (h.shape[0], h.shape[1], venc.shape[-1]))
    color_ = jax.nn.sigmoid(_mlp(jnp.concatenate([h, venc_b], axis=-1), [w2_0, w2_1, w2_2], [b2_0, b2_1, b2_2]))

    mf = mask.astype(jnp.float32)
    mask3 = jnp.broadcast_to(mf[..., None], (B, 386, 3)).reshape(B, 386 * 3)
    color2 = color_.reshape(B, 386 * 3)
    RB = 256
    color, sigma = pl.pallas_call(
        _mask_kernel,
        grid=(B // RB,),
        in_specs=[
            pl.BlockSpec((RB, 386 * 3), lambda i: (i, 0)),
            pl.BlockSpec((RB, 386), lambda i: (i, 0)),
            pl.BlockSpec((RB, 386 * 3), lambda i: (i, 0)),
            pl.BlockSpec((RB, 386), lambda i: (i, 0)),
        ],
        out_specs=(
            pl.BlockSpec((RB, 386 * 3), lambda i: (i, 0)),
            pl.BlockSpec((RB, 386), lambda i: (i, 0)),
        ),
        out_shape=(
            jax.ShapeDtypeStruct((B, 386 * 3), jnp.float32),
            jax.ShapeDtypeStruct((B, 386), jnp.float32),
        ),
    )(color2, sigma_, mask3, mf)
    return color.reshape(B, 386, 3), sigma, mask, ts[:, :-1]


# speedup vs baseline: 1.3923x; 1.3923x over previous
import functools

import jax
import jax.numpy as jnp
from jax import lax
from jax.experimental import pallas as pl
from jax.experimental.pallas import tpu as pltpu
from jax.experimental.pallas import tpu_sc as plsc

VN = 128
VD = 32
GRID = 2.0
VS = GRID / VN
XMIN = -GRID / 2.0
XMAX = GRID / 2.0
L_DIR = 4
B = 1024
M = 3 * (VN + 1)
S = M - 1
BIG = 1e10

N_SEG = B * S
N_IDX = N_SEG * 8

R = 8
NBLOCKS = B // R
RS = R * S

NC, NS = 2, 16
NW = NC * NS
IB = 1024
CH = 128
NBLK = 98
PER_W = IB * NBLK
N_PAD = PER_W * NW

C129 = VN + 1
OFFS = [dx * C129 * C129 + dy * C129 + dz
        for dx in (0, 1) for dy in (0, 1) for dz in (0, 1)]


MP = 512
NCHK = MP // 128


def _k1_body(os_ref, ds_ref, ts_ref, mask_ref, w8_ref, idx8_ref, venc_ref,
             tref, rref):
    o = os_ref[...]
    d = ds_ref[...]
    eps = 1e-9
    dsafe = jnp.where(jnp.abs(d) < eps, eps, d)

    j = lax.broadcasted_iota(jnp.int32, (R, M), 1)
    ax = j // C129
    pi = j % C129
    plane = XMIN + pi.astype(jnp.float32) * VS
    osel = jnp.where(ax == 0, o[:, 0:1], jnp.where(ax == 1, o[:, 1:2], o[:, 2:3]))
    dsel = jnp.where(ax == 0, dsafe[:, 0:1],
                     jnp.where(ax == 1, dsafe[:, 1:2], dsafe[:, 2:3]))
    t = (plane - osel) / dsel

    lo0 = jnp.minimum(t[:, 0:1], t[:, VN:VN + 1])
    lo1 = jnp.minimum(t[:, C129:C129 + 1], t[:, C129 + VN:C129 + VN + 1])
    lo2 = jnp.minimum(t[:, 2 * C129:2 * C129 + 1], t[:, 2 * C129 + VN:2 * C129 + VN + 1])
    hi0 = jnp.maximum(t[:, 0:1], t[:, VN:VN + 1])
    hi1 = jnp.maximum(t[:, C129:C129 + 1], t[:, C129 + VN:C129 + VN + 1])
    hi2 = jnp.maximum(t[:, 2 * C129:2 * C129 + 1], t[:, 2 * C129 + VN:2 * C129 + VN + 1])
    tmin = jnp.maximum(jnp.maximum(jnp.maximum(lo0, lo1), lo2), 0.0)
    tmax = jnp.minimum(jnp.minimum(hi0, hi1), hi2)

    valid = (t >= tmin) & (t <= tmax) & (tmax > tmin)
    t = jnp.where(valid, t, BIG)

    tval = jnp.concatenate([t, jnp.full((R, MP - M), BIG, jnp.float32)], axis=1)
    tref[...] = tval
    jio = lax.broadcasted_iota(jnp.int32, (R, 128, MP), 2)
    kio0 = lax.broadcasted_iota(jnp.int32, (R, 128, MP), 1)

    def rank_step(kc, racc):
        tk = tref[:, pl.ds(kc * 128, 128)]
        tk3 = tk[:, :, None]
        tj3 = tval[:, None, :]
        kio = kio0 + kc * 128
        cnt = (tk3 < tj3) | ((tk3 == tj3) & (kio < jio))
        return racc + jnp.sum(cnt.astype(jnp.float32), axis=1)

    rank = lax.fori_loop(0, NCHK, rank_step, jnp.zeros((R, MP), jnp.float32))
    rref[...] = rank
    siof = jio.astype(jnp.float32)

    def scat_step(jc, sacc):
        rk = rref[:, pl.ds(jc * 128, 128)]
        tj = tref[:, pl.ds(jc * 128, 128)]
        oh = rk[:, :, None] == siof
        return sacc + jnp.sum(jnp.where(oh, tj[:, :, None], 0.0), axis=1)

    ts = lax.fori_loop(0, NCHK, scat_step,
                       jnp.zeros((R, MP), jnp.float32))[:, :M]
    ts_ref[...] = ts

    mpts = ts < (BIG * 0.5)
    mask = mpts[:, :-1] & mpts[:, 1:]
    mask_ref[...] = mask.astype(jnp.float32)

    tin = ts[:, :-1]
    tout = ts[:, 1:]

    def coords(axis):
        oa = o[:, axis:axis + 1]
        da = d[:, axis:axis + 1]
        cin = jnp.clip((oa + tin * da - XMIN) / VS, 0.0, float(VN))
        cout = jnp.clip((oa + tout * da - XMIN) / VS, 0.0, float(VN))
        cmid = 0.5 * (cin + cout)
        cell = jnp.floor(jnp.clip(cmid, 0.0, VN - 1e-5))
        return cin, cout, cmid, cell

    cinx, coutx, cmidx, px = coords(0)
    ciny, couty, cmidy, py = coords(1)
    cinz, coutz, cmidz, pz = coords(2)

    ddx = (coutx - cinx) * VS
    ddy = (couty - ciny) * VS
    ddz = (coutz - cinz) * VS
    seg = jnp.sqrt(ddx * ddx + ddy * ddy + ddz * ddz + 1e-12)
    scale = jnp.where(mask, seg / 6.0, 0.0)

    fxs = (cinx - px, cmidx - px, coutx - px)
    fys = (ciny - py, cmidy - py, couty - py)
    fzs = (cinz - pz, cmidz - pz, coutz - pz)
    kappa = (1.0, 4.0, 1.0)

    w_parts = []
    for dx in (0, 1):
        for dy in (0, 1):
            for dz in (0, 1):
                acc = 0.0
                for q in range(3):
                    wx = fxs[q] if dx else 1.0 - fxs[q]
                    wy = fys[q] if dy else 1.0 - fys[q]
                    wz = fzs[q] if dz else 1.0 - fzs[q]
                    acc = acc + kappa[q] * (wx * wy * wz)
                w_parts.append((scale * acc)[:, :, None])
    w8 = jnp.concatenate(w_parts, axis=2)
    w8_ref[...] = w8.reshape(RS, 8)

    cid = (px.astype(jnp.int32) * C129 + py.astype(jnp.int32)) * C129 \
        + pz.astype(jnp.int32)
    idx_parts = [(cid + off)[:, :, None] for off in OFFS]
    idx8 = jnp.concatenate(idx_parts, axis=2)
    idx8_ref[...] = idx8.reshape(RS, 8)

    pieces = [d]
    for l in range(L_DIR):
        pieces.append(jnp.sin((2.0 ** l) * d))
        pieces.append(jnp.cos((2.0 ** l) * d))
    venc_ref[...] = jnp.concatenate(pieces, axis=1)


def _k1_call(os_, ds):
    return pl.pallas_call(
        _k1_body,
        grid=(NBLOCKS,),
        in_specs=[
            pl.BlockSpec((R, 3), lambda i: (i, 0)),
            pl.BlockSpec((R, 3), lambda i: (i, 0)),
        ],
        out_specs=(
            pl.BlockSpec((R, M), lambda i: (i, 0)),
            pl.BlockSpec((R, S), lambda i: (i, 0)),
            pl.BlockSpec((RS, 8), lambda i: (i, 0)),
            pl.BlockSpec((RS, 8), lambda i: (i, 0)),
            pl.BlockSpec((R, 27), lambda i: (i, 0)),
        ),
        out_shape=(
            jax.ShapeDtypeStruct((B, M), jnp.float32),
            jax.ShapeDtypeStruct((B, S), jnp.float32),
            jax.ShapeDtypeStruct((N_SEG, 8), jnp.float32),
            jax.ShapeDtypeStruct((N_SEG, 8), jnp.int32),
            jax.ShapeDtypeStruct((B, 27), jnp.float32),
        ),
        scratch_shapes=[
            pltpu.VMEM((R, MP), jnp.float32),
            pltpu.VMEM((R, MP), jnp.float32),
        ],
    )(os_, ds)


def _gather_body(idx_hbm, tab_hbm, out_hbm, idxv, rows,
                 sem_i0, sem_i1, sem_g0, sem_g1, sem_w0, sem_w1):
    wid = lax.axis_index("s") * NC + lax.axis_index("c")
    base = wid * PER_W

    sem_i = (sem_i0, sem_i1)
    sem_g = (sem_g0, sem_g1)
    sem_w = (sem_w0, sem_w1)

    def idx_dma(ib, b):
        return pltpu.make_async_copy(
            idx_hbm.at[pl.ds(base + ib * IB, IB)], idxv.at[b], sem_i[b])

    def wb_dma(ib, b):
        return pltpu.make_async_copy(
            rows.at[b], out_hbm.at[pl.ds(base + ib * IB, IB)], sem_w[b])

    idx_dma(0, 0).start()
    idx_dma(1, 1).start()

    @pl.loop(0, NBLK, step=2)
    def _group(g):
        for b in range(2):
            ib = g + b
            idx_dma(ib, b).wait()

            @pl.when(ib >= 2)
            def _():
                wb_dma(ib - 2, b).wait()

            for c in range(IB // CH):
                pltpu.make_async_copy(
                    tab_hbm.at[idxv.at[b, pl.ds(c * CH, CH)]],
                    rows.at[b, pl.ds(c * CH, CH)],
                    sem_g[b]).start()
            for c in range(IB // CH):
                pltpu.make_async_copy(
                    tab_hbm.at[idxv.at[b, pl.ds(c * CH, CH)]],
                    rows.at[b, pl.ds(c * CH, CH)],
                    sem_g[b]).wait()

            @pl.when(ib + 2 < NBLK)
            def _():
                idx_dma(ib + 2, b).start()

            wb_dma(ib, b).start()

    wb_dma(NBLK - 2, 0).wait()
    wb_dma(NBLK - 1, 1).wait()


def _gather_call(idx_pad, tab):
    mesh = plsc.VectorSubcoreMesh(core_axis_name="c", subcore_axis_name="s",
                                  num_cores=NC, num_subcores=NS)
    k = functools.partial(
        pl.kernel,
        out_type=jax.ShapeDtypeStruct((N_PAD, VD), jnp.float32),
        mesh=mesh,
        compiler_params=pltpu.CompilerParams(use_tc_tiling_on_sc=False),
        scratch_types=[
            pltpu.VMEM((2, IB), jnp.int32),
            pltpu.VMEM((2, IB, VD), jnp.float32),
            pltpu.SemaphoreType.DMA,
            pltpu.SemaphoreType.DMA,
            pltpu.SemaphoreType.DMA,
            pltpu.SemaphoreType.DMA,
            pltpu.SemaphoreType.DMA,
            pltpu.SemaphoreType.DMA,
        ],
    )(_gather_body)
    return k(idx_pad, tab)


def _k3_body(rows_ref, w8_ref, venc_ref,
             w10, b10, w11, b11, w12, b12,
             w20a, w20b, b20, w21, b21, w22, b22,
             col_ref, sig_ref):
    rows = rows_ref[...].reshape(RS, 8, VD)
    w8 = w8_ref[...]
    feat = jnp.sum(rows * w8[:, :, None], axis=1)

    x = jnp.maximum(
        jnp.dot(feat, w10[...], preferred_element_type=jnp.float32) + b10[...],
        0.0)
    x = jnp.maximum(
        jnp.dot(x, w11[...], preferred_element_type=jnp.float32) + b11[...],
        0.0)
    x = jnp.dot(x, w12[...], preferred_element_type=jnp.float32) + b12[...]

    sig = x[:, 0:1]
    h = x[:, 1:33]

    venc = venc_ref[...]
    vr = jnp.broadcast_to(venc[:, None, :], (R, S, 27)).reshape(RS, 27)

    y = (jnp.dot(h, w20a[...], preferred_element_type=jnp.float32)
         + jnp.dot(vr, w20b[...], preferred_element_type=jnp.float32)
         + b20[...])
    y = jnp.maximum(y, 0.0)
    y = jnp.maximum(
        jnp.dot(y, w21[...], preferred_element_type=jnp.float32) + b21[...],
        0.0)
    y = jnp.dot(y, w22[...], preferred_element_type=jnp.float32) + b22[...]
    color = jax.nn.sigmoid(y)

    m = (jnp.sum(w8, axis=1, keepdims=True) > 0.0).astype(jnp.float32)
    col_ref[...] = color * m
    sig_ref[...] = sig * m


def _k3_call(rows, w8, venc, w10, b10, w11, b11, w12, b12,
             w20a, w20b, b20, w21, b21, w22, b22):
    full = lambda shape: pl.BlockSpec(shape, lambda i: tuple(0 for _ in shape))
    return pl.pallas_call(
        _k3_body,
        grid=(NBLOCKS,),
        in_specs=[
            pl.BlockSpec((RS * 8, VD), lambda i: (i, 0)),
            pl.BlockSpec((RS, 8), lambda i: (i, 0)),
            pl.BlockSpec((R, 27), lambda i: (i, 0)),
            full((32, 64)), full((1, 64)),
            full((64, 64)), full((1, 64)),
            full((64, 33)), full((1, 33)),
            full((32, 64)), full((27, 64)), full((1, 64)),
            full((64, 64)), full((1, 64)),
            full((64, 3)), full((1, 3)),
        ],
        out_specs=(
            pl.BlockSpec((RS, 3), lambda i: (i, 0)),
            pl.BlockSpec((RS, 1), lambda i: (i, 0)),
        ),
        out_shape=(
            jax.ShapeDtypeStruct((N_SEG, 3), jnp.float32),
            jax.ShapeDtypeStruct((N_SEG, 1), jnp.float32),
        ),
    )(rows, w8, venc, w10, b10, w11, b11, w12, b12,
      w20a, w20b, b20, w21, b21, w22, b22)


def kernel(os, ds, voxels, w1_0, b1_0, w1_1, b1_1, w1_2, b1_2,
           w2_0, b2_0, w2_1, b2_1, w2_2, b2_2):
    ts_s, mask01, w8, idx8, venc = _k1_call(os, ds)

    idx_pad = jnp.concatenate(
        [idx8.reshape(-1), jnp.zeros((N_PAD - N_IDX,), jnp.int32)])
    tab = voxels.reshape(C129 * C129 * C129, VD)
    rows = _gather_call(idx_pad, tab)

    color_rows, sigma_rows = _k3_call(
        rows, w8, venc,
        w1_0, b1_0.reshape(1, -1), w1_1, b1_1.reshape(1, -1),
        w1_2, b1_2.reshape(1, -1),
        w2_0[:32], w2_0[32:], b2_0.reshape(1, -1),
        w2_1, b2_1.reshape(1, -1), w2_2, b2_2.reshape(1, -1))

    color = color_rows.reshape(B, S, 3)
    sigma = sigma_rows.reshape(B, S)
    mask = mask01.astype(bool)
    return color, sigma, mask, ts_s[:, :S]


# speedup vs baseline: 1.4150x; 1.0163x over previous
import functools

import jax
import jax.numpy as jnp
from jax import lax
from jax.experimental import pallas as pl
from jax.experimental.pallas import tpu as pltpu
from jax.experimental.pallas import tpu_sc as plsc

VN = 128
VD = 32
GRID = 2.0
VS = GRID / VN
XMIN = -GRID / 2.0
XMAX = GRID / 2.0
L_DIR = 4
B = 1024
M = 3 * (VN + 1)
S = M - 1
BIG = 1e10

N_SEG = B * S
N_IDX = N_SEG * 8

R = 8
NBLOCKS = B // R
RS = R * S

NC, NS = 2, 16
NW = NC * NS
CH = 128
GR = 8
IB = GR * CH
NBLK = 98
PER_W = IB * NBLK
N_PAD = PER_W * NW
N_IDXROWS = N_IDX // CH
N_PADROWS = N_PAD // CH
N_OUTROWS = N_PAD * VD // 128

C129 = VN + 1
OFFS = [dx * C129 * C129 + dy * C129 + dz
        for dx in (0, 1) for dy in (0, 1) for dz in (0, 1)]


MP = 512
NCHK = MP // 128


def _k1_body(os_ref, ds_ref, ts_ref, mask_ref, w8_ref, idx8_ref, venc_ref,
             tref, rref):
    o = os_ref[...]
    d = ds_ref[...]
    eps = 1e-9
    dsafe = jnp.where(jnp.abs(d) < eps, eps, d)

    j = lax.broadcasted_iota(jnp.int32, (R, M), 1)
    ax = j // C129
    pi = j % C129
    plane = XMIN + pi.astype(jnp.float32) * VS
    osel = jnp.where(ax == 0, o[:, 0:1], jnp.where(ax == 1, o[:, 1:2], o[:, 2:3]))
    dsel = jnp.where(ax == 0, dsafe[:, 0:1],
                     jnp.where(ax == 1, dsafe[:, 1:2], dsafe[:, 2:3]))
    t = (plane - osel) / dsel

    lo0 = jnp.minimum(t[:, 0:1], t[:, VN:VN + 1])
    lo1 = jnp.minimum(t[:, C129:C129 + 1], t[:, C129 + VN:C129 + VN + 1])
    lo2 = jnp.minimum(t[:, 2 * C129:2 * C129 + 1], t[:, 2 * C129 + VN:2 * C129 + VN + 1])
    hi0 = jnp.maximum(t[:, 0:1], t[:, VN:VN + 1])
    hi1 = jnp.maximum(t[:, C129:C129 + 1], t[:, C129 + VN:C129 + VN + 1])
    hi2 = jnp.maximum(t[:, 2 * C129:2 * C129 + 1], t[:, 2 * C129 + VN:2 * C129 + VN + 1])
    tmin = jnp.maximum(jnp.maximum(jnp.maximum(lo0, lo1), lo2), 0.0)
    tmax = jnp.minimum(jnp.minimum(hi0, hi1), hi2)

    valid = (t >= tmin) & (t <= tmax) & (tmax > tmin)
    t = jnp.where(valid, t, BIG)

    tval = jnp.concatenate([t, jnp.full((R, MP - M), BIG, jnp.float32)], axis=1)
    tref[...] = tval
    jio = lax.broadcasted_iota(jnp.int32, (R, 128, MP), 2)
    kio0 = lax.broadcasted_iota(jnp.int32, (R, 128, MP), 1)

    def rank_step(kc, racc):
        tk = tref[:, pl.ds(kc * 128, 128)]
        tk3 = tk[:, :, None]
        tj3 = tval[:, None, :]
        kio = kio0 + kc * 128
        cnt = (tk3 < tj3) | ((tk3 == tj3) & (kio < jio))
        return racc + jnp.sum(cnt.astype(jnp.float32), axis=1)

    rank = lax.fori_loop(0, NCHK, rank_step, jnp.zeros((R, MP), jnp.float32))
    rref[...] = rank
    siof = jio.astype(jnp.float32)

    def scat_step(jc, sacc):
        rk = rref[:, pl.ds(jc * 128, 128)]
        tj = tref[:, pl.ds(jc * 128, 128)]
        oh = rk[:, :, None] == siof
        return sacc + jnp.sum(jnp.where(oh, tj[:, :, None], 0.0), axis=1)

    ts = lax.fori_loop(0, NCHK, scat_step,
                       jnp.zeros((R, MP), jnp.float32))[:, :M]
    ts_ref[...] = ts

    mpts = ts < (BIG * 0.5)
    mask = mpts[:, :-1] & mpts[:, 1:]
    mask_ref[...] = mask.astype(jnp.float32)

    tin = ts[:, :-1]
    tout = ts[:, 1:]

    def coords(axis):
        oa = o[:, axis:axis + 1]
        da = d[:, axis:axis + 1]
        cin = jnp.clip((oa + tin * da - XMIN) / VS, 0.0, float(VN))
        cout = jnp.clip((oa + tout * da - XMIN) / VS, 0.0, float(VN))
        cmid = 0.5 * (cin + cout)
        cell = jnp.floor(jnp.clip(cmid, 0.0, VN - 1e-5))
        return cin, cout, cmid, cell

    cinx, coutx, cmidx, px = coords(0)
    ciny, couty, cmidy, py = coords(1)
    cinz, coutz, cmidz, pz = coords(2)

    ddx = (coutx - cinx) * VS
    ddy = (couty - ciny) * VS
    ddz = (coutz - cinz) * VS
    seg = jnp.sqrt(ddx * ddx + ddy * ddy + ddz * ddz + 1e-12)
    scale = jnp.where(mask, seg / 6.0, 0.0)

    fxs = (cinx - px, cmidx - px, coutx - px)
    fys = (ciny - py, cmidy - py, couty - py)
    fzs = (cinz - pz, cmidz - pz, coutz - pz)
    kappa = (1.0, 4.0, 1.0)

    w_parts = []
    for dx in (0, 1):
        for dy in (0, 1):
            for dz in (0, 1):
                acc = 0.0
                for q in range(3):
                    wx = fxs[q] if dx else 1.0 - fxs[q]
                    wy = fys[q] if dy else 1.0 - fys[q]
                    wz = fzs[q] if dz else 1.0 - fzs[q]
                    acc = acc + kappa[q] * (wx * wy * wz)
                w_parts.append((scale * acc)[:, :, None])
    w8 = jnp.concatenate(w_parts, axis=2)
    w8_ref[...] = w8.reshape(RS, 8)

    cid = (px.astype(jnp.int32) * C129 + py.astype(jnp.int32)) * C129 \
        + pz.astype(jnp.int32)
    idx_parts = [(cid + off)[:, :, None] for off in OFFS]
    idx8 = jnp.concatenate(idx_parts, axis=2)
    idx8_ref[...] = idx8.reshape(RS, 8)

    pieces = [d]
    for l in range(L_DIR):
        pieces.append(jnp.sin((2.0 ** l) * d))
        pieces.append(jnp.cos((2.0 ** l) * d))
    venc_ref[...] = jnp.concatenate(pieces, axis=1)


def _k1_call(os_, ds):
    return pl.pallas_call(
        _k1_body,
        grid=(NBLOCKS,),
        in_specs=[
            pl.BlockSpec((R, 3), lambda i: (i, 0)),
            pl.BlockSpec((R, 3), lambda i: (i, 0)),
        ],
        out_specs=(
            pl.BlockSpec((R, M), lambda i: (i, 0)),
            pl.BlockSpec((R, S), lambda i: (i, 0)),
            pl.BlockSpec((RS, 8), lambda i: (i, 0)),
            pl.BlockSpec((RS, 8), lambda i: (i, 0)),
            pl.BlockSpec((R, 27), lambda i: (i, 0)),
        ),
        out_shape=(
            jax.ShapeDtypeStruct((B, M), jnp.float32),
            jax.ShapeDtypeStruct((B, S), jnp.float32),
            jax.ShapeDtypeStruct((N_SEG, 8), jnp.float32),
            jax.ShapeDtypeStruct((N_SEG, 8), jnp.int32),
            jax.ShapeDtypeStruct((B, 27), jnp.float32),
        ),
        scratch_shapes=[
            pltpu.VMEM((R, MP), jnp.float32),
            pltpu.VMEM((R, MP), jnp.float32),
        ],
    )(os_, ds)


def _gather_body(idx_hbm, tab_hbm, out_hbm, idxv, rows,
                 sem_i0, sem_i1, sem_g0, sem_g1, sem_w0, sem_w1):
    wid = lax.axis_index("s") * NC + lax.axis_index("c")
    base_ir = wid * (NBLK * GR)
    base_or = wid * (PER_W * VD // 128)
    OG = IB * VD // 128

    sem_i = (sem_i0, sem_i1)
    sem_g = (sem_g0, sem_g1)
    sem_w = (sem_w0, sem_w1)

    def idx_dma(ib, b):
        return pltpu.make_async_copy(
            idx_hbm.at[pl.ds(base_ir + ib * GR, GR), :], idxv.at[b], sem_i[b])

    def wb_dma(ib, b):
        return pltpu.make_async_copy(
            rows.at[b],
            out_hbm.at[pl.ds(base_or * 4 + ib * IB, IB), :],
            sem_w[b])

    idx_dma(0, 0).start()
    idx_dma(1, 1).start()

    @pl.loop(0, NBLK, step=2)
    def _group(g):
        for b in range(2):
            ib = g + b
            idx_dma(ib, b).wait()

            @pl.when(ib >= 2)
            def _():
                wb_dma(ib - 2, b).wait()

            for c in range(GR):
                pltpu.make_async_copy(
                    tab_hbm.at[idxv.at[b, c]],
                    rows.at[b, pl.ds(c * CH, CH)],
                    sem_g[b]).start()
            for c in range(GR):
                pltpu.make_async_copy(
                    tab_hbm.at[idxv.at[b, c]],
                    rows.at[b, pl.ds(c * CH, CH)],
                    sem_g[b]).wait()

            @pl.when(ib + 2 < NBLK)
            def _():
                idx_dma(ib + 2, b).start()

            wb_dma(ib, b).start()

    wb_dma(NBLK - 2, 0).wait()
    wb_dma(NBLK - 1, 1).wait()


def _gather_call(idx_pad, tab):
    mesh = plsc.VectorSubcoreMesh(core_axis_name="c", subcore_axis_name="s",
                                  num_cores=NC, num_subcores=NS)
    k = functools.partial(
        pl.kernel,
        out_type=jax.ShapeDtypeStruct((N_PAD, VD), jnp.float32),
        mesh=mesh,
        compiler_params=pltpu.CompilerParams(use_tc_tiling_on_sc=False),
        scratch_types=[
            pltpu.VMEM((2, GR, CH), jnp.int32),
            pltpu.VMEM((2, IB, VD), jnp.float32),
            pltpu.SemaphoreType.DMA,
            pltpu.SemaphoreType.DMA,
            pltpu.SemaphoreType.DMA,
            pltpu.SemaphoreType.DMA,
            pltpu.SemaphoreType.DMA,
            pltpu.SemaphoreType.DMA,
        ],
    )(_gather_body)
    return k(idx_pad, tab)


def _k3_body(rows_ref, w8_ref, venc_ref,
             w10, b10, w11, b11, w12, b12,
             w20a, w20b, b20, w21, b21, w22, b22,
             col_ref, sig_ref):
    rows = rows_ref[...]
    w8 = w8_ref[...]
    whs = []
    for h in (0, 1):
        parts = [jnp.broadcast_to(w8[:, 4 * h + q:4 * h + q + 1], (RS, VD))
                 for q in range(4)]
        whs.append(jnp.concatenate(parts, axis=1)[:, None, :])
    wexp = jnp.concatenate(whs, axis=1).reshape(2 * RS, 128)
    prod = rows * wexp
    y = prod[:, 0:64] + prod[:, 64:128]
    z = y[:, 0:32] + y[:, 32:64]
    feat = jnp.sum(z.reshape(RS, 2, VD), axis=1)

    x = jnp.maximum(
        jnp.dot(feat, w10[...], preferred_element_type=jnp.float32) + b10[...],
        0.0)
    x = jnp.maximum(
        jnp.dot(x, w11[...], preferred_element_type=jnp.float32) + b11[...],
        0.0)
    x = jnp.dot(x, w12[...], preferred_element_type=jnp.float32) + b12[...]

    sig = x[:, 0:1]
    h = x[:, 1:33]

    venc = venc_ref[...]
    vr = jnp.broadcast_to(venc[:, None, :], (R, S, 27)).reshape(RS, 27)

    y = (jnp.dot(h, w20a[...], preferred_element_type=jnp.float32)
         + jnp.dot(vr, w20b[...], preferred_element_type=jnp.float32)
         + b20[...])
    y = jnp.maximum(y, 0.0)
    y = jnp.maximum(
        jnp.dot(y, w21[...], preferred_element_type=jnp.float32) + b21[...],
        0.0)
    y = jnp.dot(y, w22[...], preferred_element_type=jnp.float32) + b22[...]
    color = jax.nn.sigmoid(y)

    m = (jnp.sum(w8, axis=1, keepdims=True) > 0.0).astype(jnp.float32)
    col_ref[...] = color * m
    sig_ref[...] = sig * m


def _k3_call(rows, w8, venc, w10, b10, w11, b11, w12, b12,
             w20a, w20b, b20, w21, b21, w22, b22):
    full = lambda shape: pl.BlockSpec(shape, lambda i: tuple(0 for _ in shape))
    return pl.pallas_call(
        _k3_body,
        grid=(NBLOCKS,),
        in_specs=[
            pl.BlockSpec((RS * 2, 128), lambda i: (i, 0)),
            pl.BlockSpec((RS, 8), lambda i: (i, 0)),
            pl.BlockSpec((R, 27), lambda i: (i, 0)),
            full((32, 64)), full((1, 64)),
            full((64, 64)), full((1, 64)),
            full((64, 33)), full((1, 33)),
            full((32, 64)), full((27, 64)), full((1, 64)),
            full((64, 64)), full((1, 64)),
            full((64, 3)), full((1, 3)),
        ],
        out_specs=(
            pl.BlockSpec((RS, 3), lambda i: (i, 0)),
            pl.BlockSpec((RS, 1), lambda i: (i, 0)),
        ),
        out_shape=(
            jax.ShapeDtypeStruct((N_SEG, 3), jnp.float32),
            jax.ShapeDtypeStruct((N_SEG, 1), jnp.float32),
        ),
    )(rows, w8, venc, w10, b10, w11, b11, w12, b12,
      w20a, w20b, b20, w21, b21, w22, b22)


def kernel(os, ds, voxels, w1_0, b1_0, w1_1, b1_1, w1_2, b1_2,
           w2_0, b2_0, w2_1, b2_1, w2_2, b2_2):
    ts_s, mask01, w8, idx8, venc = _k1_call(os, ds)

    idx_pad = jnp.concatenate(
        [idx8.reshape(N_IDXROWS, 128),
         jnp.zeros((N_PADROWS - N_IDXROWS, 128), jnp.int32)])
    tab = voxels.reshape(C129 * C129 * C129, VD)
    rows = _gather_call(idx_pad, tab).reshape(N_OUTROWS, 128)

    color_rows, sigma_rows = _k3_call(
        rows, w8, venc,
        w1_0, b1_0.reshape(1, -1), w1_1, b1_1.reshape(1, -1),
        w1_2, b1_2.reshape(1, -1),
        w2_0[:32], w2_0[32:], b2_0.reshape(1, -1),
        w2_1, b2_1.reshape(1, -1), w2_2, b2_2.reshape(1, -1))

    color = color_rows.reshape(B, S, 3)
    sigma = sigma_rows.reshape(B, S)
    mask = mask01.astype(bool)
    return color, sigma, mask, ts_s[:, :S]
